# Initial kernel scaffold; baseline (speedup 1.0000x reference)
import jax
import jax.numpy as jnp
from jax.experimental import pallas as pl


def kernel(x, edge_index, edge_attr, batch, glbl_x, pre_n_w, pre_n_b, pre_e_w, pre_e_b, W_stack, att_stack, bias_stack, bn1_g, bn1_b, dgn_lin, dgn_g, dgn_b, ga_w0, ga_b0, ga_w1, ga_b1, ga_w2, ga_b2, post_w, post_b, out_w, out_b):
    raise NotImplementedError("write your pallas kernel here")


# speedup vs baseline: 12.8037x; 12.8037x over previous
import functools

import jax
import jax.numpy as jnp
from jax import lax
from jax.experimental import pallas as pl
from jax.experimental.pallas import tpu as pltpu
from jax.experimental.pallas import tpu_sc as plsc

N_NODES = 10000
N_GRAPHS = 100
DIM = 64
HEADS = 4
GC = 5
GROUPS = 10
LAMDA = 0.01
HD = HEADS * DIM

NPAD = 10240
NC, NS = 2, 16
NW = NC * NS
CH = 80
EB = 2000

_mesh = plsc.VectorSubcoreMesh(core_axis_name="c", subcore_axis_name="s")


def _sp(v):
    return jnp.maximum(v, 0.0) + jnp.log(1.0 + jnp.exp(-jnp.abs(v)))


def _node_prep_body(x_ref, wn_ref, bn_ref, wx_ref, ox_ref, xn_ref):
    ox = _sp(jnp.dot(x_ref[...], wn_ref[...],
                     preferred_element_type=jnp.float32) + bn_ref[...])
    ox_ref[...] = ox
    xn_ref[...] = jnp.dot(ox, wx_ref[...], preferred_element_type=jnp.float32)


def _node_prep(x, wn, bn, wx):
    return pl.pallas_call(
        _node_prep_body,
        out_shape=(jax.ShapeDtypeStruct((N_NODES, DIM), jnp.float32),
                   jax.ShapeDtypeStruct((N_NODES, HD), jnp.float32)),
    )(x, wn, bn.reshape(1, DIM), wx)


def _edge_prep_body(ea_ref, we_ref, be_ref, out_ref):
    out_ref[...] = _sp(jnp.dot(ea_ref[...], we_ref[...],
                               preferred_element_type=jnp.float32) + be_ref[...])


def _edge_prep(edge_attr, we, be):
    E, F = edge_attr.shape
    grid = (E // EB,)
    return pl.pallas_call(
        _edge_prep_body,
        grid=grid,
        in_specs=[pl.BlockSpec((EB, F), lambda i: (i, 0)),
                  pl.BlockSpec((F, DIM), lambda i: (0, 0)),
                  pl.BlockSpec((1, DIM), lambda i: (0, 0))],
        out_specs=pl.BlockSpec((EB, DIM), lambda i: (i, 0)),
        out_shape=jax.ShapeDtypeStruct((E, DIM), jnp.float32),
    )(edge_attr, we, be.reshape(1, DIM))


def _passA_body(gi_ref, gj_ref, e_ref, we_ref, al_ref, ar_ref, ap_ref):
    ew = jnp.dot(e_ref[...], we_ref[...], preferred_element_type=jnp.float32)
    oi = _sp(gi_ref[...] + ew)
    oj = _sp(gj_ref[...] + ew)
    ti = oi * al_ref[...]
    tj = oj * ar_ref[...]
    si = jnp.concatenate(
        [jnp.sum(ti[:, h * DIM:(h + 1) * DIM], axis=1, keepdims=True)
         for h in range(HEADS)], axis=1)
    sj = jnp.concatenate(
        [jnp.sum(tj[:, h * DIM:(h + 1) * DIM], axis=1, keepdims=True)
         for h in range(HEADS)], axis=1)
    ap_ref[...] = _sp(si + sj)


def _passA(gi, gj, e64, we, attl, attr_):
    E = gi.shape[0]
    grid = (E // EB,)
    return pl.pallas_call(
        _passA_body,
        grid=grid,
        in_specs=[pl.BlockSpec((EB, HD), lambda i: (i, 0)),
                  pl.BlockSpec((EB, HD), lambda i: (i, 0)),
                  pl.BlockSpec((EB, DIM), lambda i: (i, 0)),
                  pl.BlockSpec((DIM, HD), lambda i: (0, 0)),
                  pl.BlockSpec((1, HD), lambda i: (0, 0)),
                  pl.BlockSpec((1, HD), lambda i: (0, 0))],
        out_specs=pl.BlockSpec((EB, HEADS), lambda i: (i, 0)),
        out_shape=jax.ShapeDtypeStruct((E, HEADS), jnp.float32),
    )(gi, gj, e64, we, attl, attr_)


def _passB_body(ap_ref, g_ref, b_ref, gm_ref, gt_ref, z_ref):
    ap = ap_ref[...]
    m128 = jnp.mean(ap, axis=0, keepdims=True)
    mh = jnp.dot(m128, gm_ref[...],
                 preferred_element_type=jnp.float32) / 32.0
    mb = jnp.dot(mh, gt_ref[...], preferred_element_type=jnp.float32)
    d = ap - mb
    v128 = jnp.mean(d * d, axis=0, keepdims=True)
    vh = jnp.dot(v128, gm_ref[...],
                 preferred_element_type=jnp.float32) / 32.0
    vb = jnp.dot(vh, gt_ref[...], preferred_element_type=jnp.float32)
    abn = g_ref[...] * d / jnp.sqrt(vb + 1e-5) + b_ref[...]
    z_ref[...] = jnp.exp(_sp(abn))


def _passB(ap128, g128, b128, gmat, gmat_t):
    n = ap128.shape[0]
    return pl.pallas_call(
        _passB_body,
        out_shape=jax.ShapeDtypeStruct((n, 128), jnp.float32),
    )(ap128, g128, b128, gmat, gmat_t)


def _passC_body(gj_ref, e_ref, z_ref, dn_ref, we_ref, c_ref):
    ew = jnp.dot(e_ref[...], we_ref[...], preferred_element_type=jnp.float32)
    oj = _sp(gj_ref[...] + ew)
    w = z_ref[...] / (dn_ref[:, :HEADS] + 1e-16)
    acc = oj[:, 0:DIM] * w[:, 0:1]
    for h in range(1, HEADS):
        acc = acc + oj[:, h * DIM:(h + 1) * DIM] * w[:, h:h + 1]
    c_ref[...] = acc * (1.0 / HEADS)


def _passC(gj, e64, z4, dn, we):
    E = gj.shape[0]
    grid = (E // EB,)
    return pl.pallas_call(
        _passC_body,
        grid=grid,
        in_specs=[pl.BlockSpec((EB, HD), lambda i: (i, 0)),
                  pl.BlockSpec((EB, DIM), lambda i: (i, 0)),
                  pl.BlockSpec((EB, HEADS), lambda i: (i, 0)),
                  pl.BlockSpec((EB, 16), lambda i: (i, 0)),
                  pl.BlockSpec((DIM, HD), lambda i: (0, 0))],
        out_specs=pl.BlockSpec((EB, DIM), lambda i: (i, 0)),
        out_shape=jax.ShapeDtypeStruct((E, DIM), jnp.float32),
    )(gj, e64, z4, dn, we)


def _passE_body(u_ref, bias_ref, lin_ref, g_ref, b_ref, prev_ref, wxn_ref,
                ox_ref, xn_ref):
    u = u_ref[...]
    h0 = u[0, :N_NODES, :] + u[1, :N_NODES, :] + bias_ref[...]
    logits = jnp.dot(h0, lin_ref[...], preferred_element_type=jnp.float32)
    mx = jnp.max(logits, axis=1, keepdims=True)
    ez = jnp.exp(logits - mx)
    s = ez / jnp.sum(ez, axis=1, keepdims=True)
    cdims = (((0,), (0,)), ((), ()))
    mg = lax.dot_general(s, h0, cdims,
                         preferred_element_type=jnp.float32) / N_NODES
    m2 = lax.dot_general(s * s, h0 * h0, cdims,
                         preferred_element_type=jnp.float32) / N_NODES
    vg = m2 - mg * mg
    a = g_ref[...] / jnp.sqrt(vg + 1e-5)
    cvec = jnp.sum(b_ref[...] - a * mg, axis=0, keepdims=True)
    acc = jnp.dot(s, a, preferred_element_type=jnp.float32) * h0 + cvec
    newx = h0 + LAMDA * acc + prev_ref[...]
    ox_ref[...] = newx
    xn_ref[...] = jnp.dot(newx, wxn_ref[...], preferred_element_type=jnp.float32)


def _passE(u, bias, lin, g10, b10, prev, wxn):
    return pl.pallas_call(
        _passE_body,
        out_shape=(jax.ShapeDtypeStruct((N_NODES, DIM), jnp.float32),
                   jax.ShapeDtypeStruct((N_NODES, HD), jnp.float32)),
    )(u, bias.reshape(1, DIM), lin, g10, b10, prev, wxn)


def _tail_body(ox_ref, gl_ref, bc_ref, br_ref, w0a_ref, w0b_ref, b0_ref,
               w1_ref, b1_ref, w2_ref, b2_ref, pw_ref, pb_ref, ow_ref,
               ob_ref, out_ref):
    ox = ox_ref[...]
    a = _sp(jnp.dot(ox, w0a_ref[...], preferred_element_type=jnp.float32)
            + jnp.dot(gl_ref[...], w0b_ref[...],
                      preferred_element_type=jnp.float32) + b0_ref[...])
    a = _sp(jnp.dot(a, w1_ref[...], preferred_element_type=jnp.float32)
            + b1_ref[...])
    a = jnp.dot(a, w2_ref[...], preferred_element_type=jnp.float32) + b2_ref[...]
    e = jnp.exp(a)
    oh = (bc_ref[...] == lax.broadcasted_iota(
        jnp.int32, (N_NODES, N_GRAPHS), 1)).astype(jnp.float32)
    oht = (br_ref[...] == lax.broadcasted_iota(
        jnp.int32, (N_GRAPHS, N_NODES), 0)).astype(jnp.float32)
    sg = jnp.dot(oht, e, preferred_element_type=jnp.float32)
    sn = jnp.dot(oh, sg, preferred_element_type=jnp.float32)
    attw = e / (sn + 1e-16)
    pooled = jnp.dot(oht, ox * attw, preferred_element_type=jnp.float32)
    hh = _sp(jnp.dot(pooled, pw_ref[...], preferred_element_type=jnp.float32)
             + pb_ref[...])
    out_ref[...] = (jnp.dot(hh, ow_ref[...], preferred_element_type=jnp.float32)
                    + ob_ref[...])


def _tail(ox, gl, batch, w0, b0, w1, b1, w2, b2, pw, pb, ow, ob):
    bc = batch.astype(jnp.int32).reshape(N_NODES, 1)
    br = batch.astype(jnp.int32).reshape(1, N_NODES)
    return pl.pallas_call(
        _tail_body,
        out_shape=jax.ShapeDtypeStruct((N_GRAPHS, 1), jnp.float32),
    )(ox, gl, bc, br, w0[:DIM], w0[DIM:], b0.reshape(1, DIM),
      w1, b1.reshape(1, DIM), w2, b2.reshape(1, 1),
      pw, pb.reshape(1, DIM), ow, ob.reshape(1, 1))


def _sc_gather2(table, idx_i, idx_j):
    E = idx_i.shape[0]
    epw = E // NW
    nch = epw // CH

    @functools.partial(
        pl.kernel, mesh=_mesh,
        out_type=[jax.ShapeDtypeStruct((E, HD), jnp.float32),
                  jax.ShapeDtypeStruct((E, HD), jnp.float32)],
        scratch_types=[pltpu.VMEM((CH,), jnp.int32),
                       pltpu.VMEM((CH,), jnp.int32),
                       pltpu.VMEM((CH, HD), jnp.float32),
                       pltpu.VMEM((CH, HD), jnp.float32),
                       pltpu.SemaphoreType.DMA,
                       pltpu.SemaphoreType.DMA])
    def k(tab, ii, ij, gi, gj, iv, jv, ri, rj, s1, s2):
        wid = lax.axis_index("s") * NC + lax.axis_index("c")
        base = wid * epw

        def body(t, carry):
            o = base + t * CH
            pltpu.sync_copy(ii.at[pl.ds(o, CH)], iv)
            pltpu.sync_copy(ij.at[pl.ds(o, CH)], jv)
            ci = pltpu.async_copy(tab.at[iv], ri, s1)
            cj = pltpu.async_copy(tab.at[jv], rj, s2)
            ci.wait()
            pltpu.sync_copy(ri, gi.at[pl.ds(o, CH)])
            cj.wait()
            pltpu.sync_copy(rj, gj.at[pl.ds(o, CH)])
            return carry

        lax.fori_loop(0, nch, body, 0)

    return k(table, idx_i, idx_j)


def _sc_gather_sum(p0, p1, idx_i):
    E = idx_i.shape[0]
    epw = E // NW
    nch = epw // CH

    @functools.partial(
        pl.kernel, mesh=_mesh,
        out_type=jax.ShapeDtypeStruct((E, 16), jnp.float32),
        compiler_params=pltpu.CompilerParams(use_tc_tiling_on_sc=False),
        scratch_types=[pltpu.VMEM((CH,), jnp.int32),
                       pltpu.VMEM((CH, 16), jnp.float32),
                       pltpu.VMEM((CH, 16), jnp.float32),
                       pltpu.SemaphoreType.DMA,
                       pltpu.SemaphoreType.DMA])
    def k(t0, t1, ii, dn, iv, r0, r1, s1, s2):
        wid = lax.axis_index("s") * NC + lax.axis_index("c")
        base = wid * epw

        def body(t, carry):
            o = base + t * CH
            pltpu.sync_copy(ii.at[pl.ds(o, CH)], iv)
            c0 = pltpu.async_copy(t0.at[iv], r0, s1)
            c1 = pltpu.async_copy(t1.at[iv], r1, s2)
            c0.wait()
            c1.wait()
            for q in range(CH):
                r0[q, :] = r0[q, :] + r1[q, :]
            pltpu.sync_copy(r0, dn.at[pl.ds(o, CH)])
            return carry

        lax.fori_loop(0, nch, body, 0)

    return k(p0, p1, idx_i)


def _sc_scatter(vals, idx_i, d):
    E = idx_i.shape[0]
    epw = E // NW
    nch = epw // CH
    rps = NPAD // NS

    @functools.partial(
        pl.kernel, mesh=_mesh,
        out_type=jax.ShapeDtypeStruct((NC, NPAD, d), jnp.float32),
        scratch_types=[pltpu.VMEM((CH,), jnp.int32),
                       pltpu.VMEM((CH, d), jnp.float32),
                       pltpu.VMEM((CH, d), jnp.float32),
                       pltpu.VMEM_SHARED((NPAD, d), jnp.float32)])
    def k(v, ii, out, iv, rows, zb, acc):
        c = lax.axis_index("c")
        s = lax.axis_index("s")
        wid = s * NC + c
        for q in range(CH):
            for j in range(d // 16):
                zb[q, pl.ds(j * 16, 16)] = jnp.zeros((16,), jnp.float32)

        def zbody(t, carry):
            pltpu.sync_copy(zb, acc.at[pl.ds(s * rps + t * CH, CH)])
            return carry

        lax.fori_loop(0, rps // CH, zbody, 0)
        plsc.subcore_barrier()
        base = wid * epw

        def body(t, carry):
            o = base + t * CH
            pltpu.sync_copy(ii.at[pl.ds(o, CH)], iv)
            pltpu.sync_copy(v.at[pl.ds(o, CH)], rows)
            pltpu.sync_copy(rows, acc.at[iv], add=True)
            return carry

        lax.fori_loop(0, nch, body, 0)
        plsc.subcore_barrier()

        def dbody(t, carry):
            o = s * rps + t * CH
            pltpu.sync_copy(acc.at[pl.ds(o, CH)], zb)
            pltpu.sync_copy(zb, out.at[c, pl.ds(o, CH)])
            return carry

        lax.fori_loop(0, rps // CH, dbody, 0)

    return k(vals, idx_i)


def kernel(x, edge_index, edge_attr, batch, glbl_x, pre_n_w, pre_n_b,
           pre_e_w, pre_e_b, W_stack, att_stack, bias_stack, bn1_g, bn1_b,
           dgn_lin, dgn_g, dgn_b, ga_w0, ga_b0, ga_w1, ga_b1, ga_w2, ga_b2,
           post_w, post_b, out_w, out_b):
    idx_i = edge_index[0].astype(jnp.int32)
    idx_j = edge_index[1].astype(jnp.int32)
    E = idx_i.shape[0]

    gmat = (jnp.arange(128)[:, None] % HEADS
            == jnp.arange(HEADS)[None, :]).astype(jnp.float32)
    gmat_t = gmat.T

    ox, xn = _node_prep(x, pre_n_w, pre_n_b, W_stack[0][:DIM])
    e64 = _edge_prep(edge_attr, pre_e_w, pre_e_b)

    for l in range(GC):
        we = W_stack[l][DIM:]
        attl = att_stack[l][0][:, :DIM].reshape(1, HD)
        attr_ = att_stack[l][0][:, DIM:].reshape(1, HD)
        g128 = jnp.tile(bn1_g[l].reshape(1, HEADS), (1, 128 // HEADS))
        b128 = jnp.tile(bn1_b[l].reshape(1, HEADS), (1, 128 // HEADS))

        gi, gj = _sc_gather2(xn, idx_i, idx_j)
        ap = _passA(gi, gj, e64, we, attl, attr_)
        z128 = _passB(ap.reshape(E * HEADS // 128, 128), g128, b128,
                      gmat, gmat_t)
        z4 = z128.reshape(E, HEADS)
        z16 = jnp.pad(z4, ((0, 0), (0, 16 - HEADS)))
        dpart = _sc_scatter(z16, idx_i, 16)
        dn = _sc_gather_sum(dpart[0], dpart[1], idx_i)
        c = _passC(gj, e64, z4, dn, we)
        upart = _sc_scatter(c, idx_i, DIM)
        wxn = W_stack[min(l + 1, GC - 1)][:DIM]
        ox, xn = _passE(upart, bias_stack[l], dgn_lin[l],
                        dgn_g[l].reshape(GROUPS, DIM),
                        dgn_b[l].reshape(GROUPS, DIM), ox, wxn)

    out = _tail(ox, glbl_x, batch, ga_w0, ga_b0, ga_w1, ga_b1, ga_w2, ga_b2,
                post_w, post_b, out_w, out_b)
    return out.reshape(-1)


# speedup vs baseline: 13.0970x; 1.0229x over previous
import functools

import jax
import jax.numpy as jnp
from jax import lax
from jax.experimental import pallas as pl
from jax.experimental.pallas import tpu as pltpu
from jax.experimental.pallas import tpu_sc as plsc

N_NODES = 10000
N_GRAPHS = 100
DIM = 64
HEADS = 4
GC = 5
GROUPS = 10
LAMDA = 0.01
HD = HEADS * DIM

NPAD = 10240
NC, NS = 2, 16
NW = NC * NS
EPW = 320000 // NW
EB = 2000

_mesh = plsc.VectorSubcoreMesh(core_axis_name="c", subcore_axis_name="s")


def _sp(v):
    return jnp.maximum(v, 0.0) + jnp.log(1.0 + jnp.exp(-jnp.abs(v)))


def _node_prep_body(x_ref, wn_ref, bn_ref, wx_ref, ox_ref, xn_ref):
    ox = _sp(jnp.dot(x_ref[...], wn_ref[...],
                     preferred_element_type=jnp.float32) + bn_ref[...])
    ox_ref[...] = ox
    xn_ref[...] = jnp.dot(ox, wx_ref[...], preferred_element_type=jnp.float32)


def _node_prep(x, wn, bn, wx):
    return pl.pallas_call(
        _node_prep_body,
        out_shape=(jax.ShapeDtypeStruct((N_NODES, DIM), jnp.float32),
                   jax.ShapeDtypeStruct((N_NODES, HD), jnp.float32)),
    )(x, wn, bn.reshape(1, DIM), wx)


def _edge_prep_body(ea_ref, we_ref, be_ref, out_ref):
    out_ref[...] = _sp(jnp.dot(ea_ref[...], we_ref[...],
                               preferred_element_type=jnp.float32) + be_ref[...])


def _edge_prep(edge_attr, we, be):
    E, F = edge_attr.shape
    grid = (E // EB,)
    return pl.pallas_call(
        _edge_prep_body,
        grid=grid,
        in_specs=[pl.BlockSpec((EB, F), lambda i: (i, 0)),
                  pl.BlockSpec((F, DIM), lambda i: (0, 0)),
                  pl.BlockSpec((1, DIM), lambda i: (0, 0))],
        out_specs=pl.BlockSpec((EB, DIM), lambda i: (i, 0)),
        out_shape=jax.ShapeDtypeStruct((E, DIM), jnp.float32),
    )(edge_attr, we, be.reshape(1, DIM))


def _passA_body(gi_ref, gj_ref, e_ref, we_ref, al_ref, ar_ref, ap_ref):
    ew = jnp.dot(e_ref[...], we_ref[...], preferred_element_type=jnp.float32)
    oi = _sp(gi_ref[...] + ew)
    oj = _sp(gj_ref[...] + ew)
    ti = oi * al_ref[...]
    tj = oj * ar_ref[...]
    si = jnp.concatenate(
        [jnp.sum(ti[:, h * DIM:(h + 1) * DIM], axis=1, keepdims=True)
         for h in range(HEADS)], axis=1)
    sj = jnp.concatenate(
        [jnp.sum(tj[:, h * DIM:(h + 1) * DIM], axis=1, keepdims=True)
         for h in range(HEADS)], axis=1)
    ap_ref[...] = _sp(si + sj)


def _passA(gi, gj, e64, we, attl, attr_):
    E = gi.shape[0]
    grid = (E // EB,)
    return pl.pallas_call(
        _passA_body,
        grid=grid,
        in_specs=[pl.BlockSpec((EB, HD), lambda i: (i, 0)),
                  pl.BlockSpec((EB, HD), lambda i: (i, 0)),
                  pl.BlockSpec((EB, DIM), lambda i: (i, 0)),
                  pl.BlockSpec((DIM, HD), lambda i: (0, 0)),
                  pl.BlockSpec((1, HD), lambda i: (0, 0)),
                  pl.BlockSpec((1, HD), lambda i: (0, 0))],
        out_specs=pl.BlockSpec((EB, HEADS), lambda i: (i, 0)),
        out_shape=jax.ShapeDtypeStruct((E, HEADS), jnp.float32),
    )(gi, gj, e64, we, attl, attr_)


def _passB_body(ap_ref, g_ref, b_ref, gm_ref, gt_ref, z_ref):
    ap = ap_ref[...]
    m128 = jnp.mean(ap, axis=0, keepdims=True)
    mh = jnp.dot(m128, gm_ref[...],
                 preferred_element_type=jnp.float32) / 32.0
    mb = jnp.dot(mh, gt_ref[...], preferred_element_type=jnp.float32)
    d = ap - mb
    v128 = jnp.mean(d * d, axis=0, keepdims=True)
    vh = jnp.dot(v128, gm_ref[...],
                 preferred_element_type=jnp.float32) / 32.0
    vb = jnp.dot(vh, gt_ref[...], preferred_element_type=jnp.float32)
    abn = g_ref[...] * d / jnp.sqrt(vb + 1e-5) + b_ref[...]
    z_ref[...] = jnp.exp(_sp(abn))


def _passB(ap128, g128, b128, gmat, gmat_t):
    n = ap128.shape[0]
    return pl.pallas_call(
        _passB_body,
        out_shape=jax.ShapeDtypeStruct((n, 128), jnp.float32),
    )(ap128, g128, b128, gmat, gmat_t)


def _passC_body(gj_ref, e_ref, z_ref, dn_ref, we_ref, c_ref):
    ew = jnp.dot(e_ref[...], we_ref[...], preferred_element_type=jnp.float32)
    oj = _sp(gj_ref[...] + ew)
    w = z_ref[...] / (dn_ref[:, :HEADS] + 1e-16)
    acc = oj[:, 0:DIM] * w[:, 0:1]
    for h in range(1, HEADS):
        acc = acc + oj[:, h * DIM:(h + 1) * DIM] * w[:, h:h + 1]
    c_ref[...] = acc * (1.0 / HEADS)


def _passC(gj, e64, z4, dn, we):
    E = gj.shape[0]
    grid = (E // EB,)
    return pl.pallas_call(
        _passC_body,
        grid=grid,
        in_specs=[pl.BlockSpec((EB, HD), lambda i: (i, 0)),
                  pl.BlockSpec((EB, DIM), lambda i: (i, 0)),
                  pl.BlockSpec((EB, HEADS), lambda i: (i, 0)),
                  pl.BlockSpec((EB, DIM), lambda i: (i, 0)),
                  pl.BlockSpec((DIM, HD), lambda i: (0, 0))],
        out_specs=pl.BlockSpec((EB, DIM), lambda i: (i, 0)),
        out_shape=jax.ShapeDtypeStruct((E, DIM), jnp.float32),
    )(gj, e64, z4, dn, we)


def _passE_body(u_ref, bias_ref, lin_ref, g_ref, b_ref, prev_ref, wxn_ref,
                ox_ref, xn_ref):
    u = u_ref[...]
    h0 = u[0, :N_NODES, :] + u[1, :N_NODES, :] + bias_ref[...]
    logits = jnp.dot(h0, lin_ref[...], preferred_element_type=jnp.float32)
    mx = jnp.max(logits, axis=1, keepdims=True)
    ez = jnp.exp(logits - mx)
    s = ez / jnp.sum(ez, axis=1, keepdims=True)
    cdims = (((0,), (0,)), ((), ()))
    mg = lax.dot_general(s, h0, cdims,
                         preferred_element_type=jnp.float32) / N_NODES
    m2 = lax.dot_general(s * s, h0 * h0, cdims,
                         preferred_element_type=jnp.float32) / N_NODES
    vg = m2 - mg * mg
    a = g_ref[...] / jnp.sqrt(vg + 1e-5)
    cvec = jnp.sum(b_ref[...] - a * mg, axis=0, keepdims=True)
    acc = jnp.dot(s, a, preferred_element_type=jnp.float32) * h0 + cvec
    newx = h0 + LAMDA * acc + prev_ref[...]
    ox_ref[...] = newx
    xn_ref[...] = jnp.dot(newx, wxn_ref[...], preferred_element_type=jnp.float32)


def _passE(u, bias, lin, g10, b10, prev, wxn):
    return pl.pallas_call(
        _passE_body,
        out_shape=(jax.ShapeDtypeStruct((N_NODES, DIM), jnp.float32),
                   jax.ShapeDtypeStruct((N_NODES, HD), jnp.float32)),
    )(u, bias.reshape(1, DIM), lin, g10, b10, prev, wxn)


def _tail_body(ox_ref, gl_ref, bc_ref, br_ref, w0a_ref, w0b_ref, b0_ref,
               w1_ref, b1_ref, w2_ref, b2_ref, pw_ref, pb_ref, ow_ref,
               ob_ref, out_ref):
    ox = ox_ref[...]
    a = _sp(jnp.dot(ox, w0a_ref[...], preferred_element_type=jnp.float32)
            + jnp.dot(gl_ref[...], w0b_ref[...],
                      preferred_element_type=jnp.float32) + b0_ref[...])
    a = _sp(jnp.dot(a, w1_ref[...], preferred_element_type=jnp.float32)
            + b1_ref[...])
    a = jnp.dot(a, w2_ref[...], preferred_element_type=jnp.float32) + b2_ref[...]
    e = jnp.exp(a)
    oh = (bc_ref[...] == lax.broadcasted_iota(
        jnp.int32, (N_NODES, N_GRAPHS), 1)).astype(jnp.float32)
    oht = (br_ref[...] == lax.broadcasted_iota(
        jnp.int32, (N_GRAPHS, N_NODES), 0)).astype(jnp.float32)
    sg = jnp.dot(oht, e, preferred_element_type=jnp.float32)
    sn = jnp.dot(oh, sg, preferred_element_type=jnp.float32)
    attw = e / (sn + 1e-16)
    pooled = jnp.dot(oht, ox * attw, preferred_element_type=jnp.float32)
    hh = _sp(jnp.dot(pooled, pw_ref[...], preferred_element_type=jnp.float32)
             + pb_ref[...])
    out_ref[...] = (jnp.dot(hh, ow_ref[...], preferred_element_type=jnp.float32)
                    + ob_ref[...])


def _tail(ox, gl, batch, w0, b0, w1, b1, w2, b2, pw, pb, ow, ob):
    bc = batch.astype(jnp.int32).reshape(N_NODES, 1)
    br = batch.astype(jnp.int32).reshape(1, N_NODES)
    return pl.pallas_call(
        _tail_body,
        out_shape=jax.ShapeDtypeStruct((N_GRAPHS, 1), jnp.float32),
    )(ox, gl, bc, br, w0[:DIM], w0[DIM:], b0.reshape(1, DIM),
      w1, b1.reshape(1, DIM), w2, b2.reshape(1, 1),
      pw, pb.reshape(1, DIM), ow, ob.reshape(1, 1))


def _sc_gather2(table, idx_i, idx_j):
    E = idx_i.shape[0]
    ch = 80
    nch = EPW // ch

    @functools.partial(
        pl.kernel, mesh=_mesh,
        out_type=[jax.ShapeDtypeStruct((E, HD), jnp.float32),
                  jax.ShapeDtypeStruct((E, HD), jnp.float32)],
        scratch_types=[pltpu.VMEM((ch,), jnp.int32),
                       pltpu.VMEM((ch,), jnp.int32),
                       pltpu.VMEM((ch, HD), jnp.float32),
                       pltpu.VMEM((ch, HD), jnp.float32),
                       pltpu.SemaphoreType.DMA,
                       pltpu.SemaphoreType.DMA])
    def k(tab, ii, ij, gi, gj, iv, jv, ri, rj, s1, s2):
        wid = lax.axis_index("s") * NC + lax.axis_index("c")
        base = wid * EPW

        def body(t, carry):
            o = base + t * ch
            pltpu.sync_copy(ii.at[pl.ds(o, ch)], iv)
            pltpu.sync_copy(ij.at[pl.ds(o, ch)], jv)
            ci = pltpu.async_copy(tab.at[iv], ri, s1)
            cj = pltpu.async_copy(tab.at[jv], rj, s2)
            ci.wait()
            pltpu.sync_copy(ri, gi.at[pl.ds(o, ch)])
            cj.wait()
            pltpu.sync_copy(rj, gj.at[pl.ds(o, ch)])
            return carry

        lax.fori_loop(0, nch, body, 0)

    return k(table, idx_i, idx_j)


def _combine16_body(u_ref, o_ref):
    u = u_ref[...]
    s = u[0] + u[1]
    o_ref[...] = jnp.concatenate(
        [s, jnp.zeros((NPAD, DIM - 16), jnp.float32)], axis=1)


def _combine16(dpart):
    return pl.pallas_call(
        _combine16_body,
        out_shape=jax.ShapeDtypeStruct((NPAD, DIM), jnp.float32),
    )(dpart)


def _sc_gather16(table, idxp):
    E = NW * EPW
    ch = 400
    sub = ch // 80
    nch = EPW // ch

    @functools.partial(
        pl.kernel, mesh=_mesh,
        out_type=jax.ShapeDtypeStruct((E, DIM), jnp.float32),
        compiler_params=pltpu.CompilerParams(use_tc_tiling_on_sc=False),
        scratch_types=[pltpu.VMEM((128, 80), jnp.int32),
                       pltpu.VMEM((ch, DIM), jnp.float32),
                       pltpu.SemaphoreType.DMA])
    def k(t0, ii, dn, iv, rows, s1):
        wid = lax.axis_index("s") * NC + lax.axis_index("c")
        base = wid * EPW
        pltpu.sync_copy(ii.at[pl.ds(wid * 128, 128)], iv)

        for t in range(nch):
            o = base + t * ch
            hs = [pltpu.async_copy(t0.at[iv.at[t * sub + j]],
                                   rows.at[pl.ds(j * 80, 80)], s1)
                  for j in range(sub)]
            for h in hs:
                h.wait()
            pltpu.sync_copy(rows, dn.at[pl.ds(o, ch)])

    return k(table, idxp)


def _sc_scatter(vals, idx_i, d):
    E = idx_i.shape[0]
    ch = 80
    nch = EPW // ch
    rps = NPAD // NS

    @functools.partial(
        pl.kernel, mesh=_mesh,
        out_type=jax.ShapeDtypeStruct((NC, NPAD, d), jnp.float32),
        scratch_types=[pltpu.VMEM((ch,), jnp.int32),
                       pltpu.VMEM((ch, d), jnp.float32),
                       pltpu.VMEM((ch, d), jnp.float32),
                       pltpu.VMEM_SHARED((NPAD, d), jnp.float32)])
    def k(v, ii, out, iv, rows, zb, acc):
        c = lax.axis_index("c")
        s = lax.axis_index("s")
        wid = s * NC + c
        for q in range(ch):
            for j in range(d // 16):
                zb[q, pl.ds(j * 16, 16)] = jnp.zeros((16,), jnp.float32)

        def zbody(t, carry):
            pltpu.sync_copy(zb, acc.at[pl.ds(s * rps + t * ch, ch)])
            return carry

        lax.fori_loop(0, rps // ch, zbody, 0)
        plsc.subcore_barrier()
        base = wid * EPW

        def body(t, carry):
            o = base + t * ch
            pltpu.sync_copy(ii.at[pl.ds(o, ch)], iv)
            pltpu.sync_copy(v.at[pl.ds(o, ch)], rows)
            pltpu.sync_copy(rows, acc.at[iv], add=True)
            return carry

        lax.fori_loop(0, nch, body, 0)
        plsc.subcore_barrier()

        def dbody(t, carry):
            o = s * rps + t * ch
            pltpu.sync_copy(acc.at[pl.ds(o, ch)], zb)
            pltpu.sync_copy(zb, out.at[c, pl.ds(o, ch)])
            return carry

        lax.fori_loop(0, rps // ch, dbody, 0)

    return k(vals, idx_i)


def kernel(x, edge_index, edge_attr, batch, glbl_x, pre_n_w, pre_n_b,
           pre_e_w, pre_e_b, W_stack, att_stack, bias_stack, bn1_g, bn1_b,
           dgn_lin, dgn_g, dgn_b, ga_w0, ga_b0, ga_w1, ga_b1, ga_w2, ga_b2,
           post_w, post_b, out_w, out_b):
    idx_i = edge_index[0].astype(jnp.int32)
    idx_j = edge_index[1].astype(jnp.int32)
    E = idx_i.shape[0]

    def _padidx(ix):
        r = ix.reshape(NW, EPW // 80, 80)
        p = jnp.full((NW, 128 - EPW // 80, 80), NPAD - 1, jnp.int32)
        return jnp.concatenate([r, p], axis=1).reshape(NW * 128, 80)

    idxp_i = _padidx(idx_i)
    idxp_j = _padidx(idx_j)
    zeros64 = jnp.zeros((80, DIM), jnp.float32)

    gmat = (jnp.arange(128)[:, None] % HEADS
            == jnp.arange(HEADS)[None, :]).astype(jnp.float32)
    gmat_t = gmat.T

    ox, xn = _node_prep(x, pre_n_w, pre_n_b, W_stack[0][:DIM])
    e64 = _edge_prep(edge_attr, pre_e_w, pre_e_b)

    for l in range(GC):
        we = W_stack[l][DIM:]
        attl = att_stack[l][0][:, :DIM].reshape(1, HD)
        attr_ = att_stack[l][0][:, DIM:].reshape(1, HD)
        g128 = jnp.tile(bn1_g[l].reshape(1, HEADS), (1, 128 // HEADS))
        b128 = jnp.tile(bn1_b[l].reshape(1, HEADS), (1, 128 // HEADS))

        gi, gj = _sc_gather2(xn, idx_i, idx_j)
        ap = _passA(gi, gj, e64, we, attl, attr_)
        z128 = _passB(ap.reshape(E * HEADS // 128, 128), g128, b128,
                      gmat, gmat_t)
        z4 = z128.reshape(E, HEADS)
        z16 = jnp.pad(z4, ((0, 0), (0, 16 - HEADS)))
        dpart = _sc_scatter(z16, idx_i, 16)
        dn = _sc_gather16(_combine16(dpart), idxp_i)
        c = _passC(gj, e64, z4, dn, we)
        upart = _sc_scatter(c, idx_i, DIM)
        wxn = W_stack[min(l + 1, GC - 1)][:DIM]
        ox, xn = _passE(upart, bias_stack[l], dgn_lin[l],
                        dgn_g[l].reshape(GROUPS, DIM),
                        dgn_b[l].reshape(GROUPS, DIM), ox, wxn)

    out = _tail(ox, glbl_x, batch, ga_w0, ga_b0, ga_w1, ga_b1, ga_w2, ga_b2,
                post_w, post_b, out_w, out_b)
    return out.reshape(-1)


# speedup vs baseline: 13.0991x; 1.0002x over previous
import functools

import jax
import jax.numpy as jnp
from jax import lax
from jax.experimental import pallas as pl
from jax.experimental.pallas import tpu as pltpu
from jax.experimental.pallas import tpu_sc as plsc

N_NODES = 10000
N_GRAPHS = 100
DIM = 64
HEADS = 4
GC = 5
GROUPS = 10
LAMDA = 0.01
HD = HEADS * DIM

NPAD = 10240
NC, NS = 2, 16
NW = NC * NS
EPW = 320000 // NW
EB = 2000

_mesh = plsc.VectorSubcoreMesh(core_axis_name="c", subcore_axis_name="s")


def _sp(v):
    return jnp.maximum(v, 0.0) + jnp.log(1.0 + jnp.exp(-jnp.abs(v)))


def _node_prep_body(x_ref, wn_ref, bn_ref, wx_ref, ox_ref, xn_ref):
    ox = _sp(jnp.dot(x_ref[...], wn_ref[...],
                     preferred_element_type=jnp.float32) + bn_ref[...])
    ox_ref[...] = ox
    xn_ref[...] = jnp.dot(ox, wx_ref[...], preferred_element_type=jnp.float32)


def _node_prep(x, wn, bn, wx):
    return pl.pallas_call(
        _node_prep_body,
        out_shape=(jax.ShapeDtypeStruct((N_NODES, DIM), jnp.float32),
                   jax.ShapeDtypeStruct((N_NODES, HD), jnp.float32)),
    )(x, wn, bn.reshape(1, DIM), wx)


def _edge_prep_body(ea_ref, we_ref, be_ref, out_ref):
    out_ref[...] = _sp(jnp.dot(ea_ref[...], we_ref[...],
                               preferred_element_type=jnp.float32) + be_ref[...])


def _edge_prep(edge_attr, we, be):
    E, F = edge_attr.shape
    grid = (E // EB,)
    return pl.pallas_call(
        _edge_prep_body,
        grid=grid,
        in_specs=[pl.BlockSpec((EB, F), lambda i: (i, 0)),
                  pl.BlockSpec((F, DIM), lambda i: (0, 0)),
                  pl.BlockSpec((1, DIM), lambda i: (0, 0))],
        out_specs=pl.BlockSpec((EB, DIM), lambda i: (i, 0)),
        out_shape=jax.ShapeDtypeStruct((E, DIM), jnp.float32),
    )(edge_attr, we, be.reshape(1, DIM))


def _passA_body(gi_ref, gj_ref, e_ref, we_ref, al_ref, ar_ref, ap_ref):
    ew = jnp.dot(e_ref[...], we_ref[...], preferred_element_type=jnp.float32)
    oi = _sp(gi_ref[...] + ew)
    oj = _sp(gj_ref[...] + ew)
    ti = oi * al_ref[...]
    tj = oj * ar_ref[...]
    si = jnp.concatenate(
        [jnp.sum(ti[:, h * DIM:(h + 1) * DIM], axis=1, keepdims=True)
         for h in range(HEADS)], axis=1)
    sj = jnp.concatenate(
        [jnp.sum(tj[:, h * DIM:(h + 1) * DIM], axis=1, keepdims=True)
         for h in range(HEADS)], axis=1)
    ap_ref[...] = _sp(si + sj)


def _passA(gi, gj, e64, we, attl, attr_):
    E = gi.shape[0]
    grid = (E // EB,)
    return pl.pallas_call(
        _passA_body,
        grid=grid,
        in_specs=[pl.BlockSpec((EB, HD), lambda i: (i, 0)),
                  pl.BlockSpec((EB, HD), lambda i: (i, 0)),
                  pl.BlockSpec((EB, DIM), lambda i: (i, 0)),
                  pl.BlockSpec((DIM, HD), lambda i: (0, 0)),
                  pl.BlockSpec((1, HD), lambda i: (0, 0)),
                  pl.BlockSpec((1, HD), lambda i: (0, 0))],
        out_specs=pl.BlockSpec((EB, HEADS), lambda i: (i, 0)),
        out_shape=jax.ShapeDtypeStruct((E, HEADS), jnp.float32),
    )(gi, gj, e64, we, attl, attr_)


def _passB_body(ap_ref, g_ref, b_ref, gm_ref, gt_ref, z_ref):
    ap = ap_ref[...]
    m128 = jnp.mean(ap, axis=0, keepdims=True)
    mh = jnp.dot(m128, gm_ref[...],
                 preferred_element_type=jnp.float32) / 32.0
    mb = jnp.dot(mh, gt_ref[...], preferred_element_type=jnp.float32)
    d = ap - mb
    v128 = jnp.mean(d * d, axis=0, keepdims=True)
    vh = jnp.dot(v128, gm_ref[...],
                 preferred_element_type=jnp.float32) / 32.0
    vb = jnp.dot(vh, gt_ref[...], preferred_element_type=jnp.float32)
    abn = g_ref[...] * d / jnp.sqrt(vb + 1e-5) + b_ref[...]
    z_ref[...] = jnp.exp(_sp(abn))


def _passB(ap128, g128, b128, gmat, gmat_t):
    n = ap128.shape[0]
    return pl.pallas_call(
        _passB_body,
        out_shape=jax.ShapeDtypeStruct((n, 128), jnp.float32),
    )(ap128, g128, b128, gmat, gmat_t)


def _passC_body(gj_ref, e_ref, z_ref, dn_ref, we_ref, c_ref):
    ew = jnp.dot(e_ref[...], we_ref[...], preferred_element_type=jnp.float32)
    oj = _sp(gj_ref[...] + ew)
    w = z_ref[...] / (dn_ref[:, :HEADS] + 1e-16)
    acc = oj[:, 0:DIM] * w[:, 0:1]
    for h in range(1, HEADS):
        acc = acc + oj[:, h * DIM:(h + 1) * DIM] * w[:, h:h + 1]
    c_ref[...] = acc * (1.0 / HEADS)


def _passC(gj, e64, z4, dn, we):
    E = gj.shape[0]
    grid = (E // EB,)
    return pl.pallas_call(
        _passC_body,
        grid=grid,
        in_specs=[pl.BlockSpec((EB, HD), lambda i: (i, 0)),
                  pl.BlockSpec((EB, DIM), lambda i: (i, 0)),
                  pl.BlockSpec((EB, HEADS), lambda i: (i, 0)),
                  pl.BlockSpec((EB, DIM), lambda i: (i, 0)),
                  pl.BlockSpec((DIM, HD), lambda i: (0, 0))],
        out_specs=pl.BlockSpec((EB, DIM), lambda i: (i, 0)),
        out_shape=jax.ShapeDtypeStruct((E, DIM), jnp.float32),
    )(gj, e64, z4, dn, we)


def _passE_body(u_ref, bias_ref, lin_ref, g_ref, b_ref, prev_ref, wxn_ref,
                ox_ref, xn_ref):
    u = u_ref[...]
    h0 = u[0, :N_NODES, :] + u[1, :N_NODES, :] + bias_ref[...]
    logits = jnp.dot(h0, lin_ref[...], preferred_element_type=jnp.float32)
    mx = jnp.max(logits, axis=1, keepdims=True)
    ez = jnp.exp(logits - mx)
    s = ez / jnp.sum(ez, axis=1, keepdims=True)
    cdims = (((0,), (0,)), ((), ()))
    mg = lax.dot_general(s, h0, cdims,
                         preferred_element_type=jnp.float32) / N_NODES
    m2 = lax.dot_general(s * s, h0 * h0, cdims,
                         preferred_element_type=jnp.float32) / N_NODES
    vg = m2 - mg * mg
    a = g_ref[...] / jnp.sqrt(vg + 1e-5)
    cvec = jnp.sum(b_ref[...] - a * mg, axis=0, keepdims=True)
    acc = jnp.dot(s, a, preferred_element_type=jnp.float32) * h0 + cvec
    newx = h0 + LAMDA * acc + prev_ref[...]
    ox_ref[...] = newx
    xn_ref[...] = jnp.dot(newx, wxn_ref[...], preferred_element_type=jnp.float32)


def _passE(u, bias, lin, g10, b10, prev, wxn):
    return pl.pallas_call(
        _passE_body,
        out_shape=(jax.ShapeDtypeStruct((N_NODES, DIM), jnp.float32),
                   jax.ShapeDtypeStruct((N_NODES, HD), jnp.float32)),
    )(u, bias.reshape(1, DIM), lin, g10, b10, prev, wxn)


def _tail_body(ox_ref, gl_ref, bc_ref, br_ref, w0a_ref, w0b_ref, b0_ref,
               w1_ref, b1_ref, w2_ref, b2_ref, pw_ref, pb_ref, ow_ref,
               ob_ref, out_ref):
    ox = ox_ref[...]
    a = _sp(jnp.dot(ox, w0a_ref[...], preferred_element_type=jnp.float32)
            + jnp.dot(gl_ref[...], w0b_ref[...],
                      preferred_element_type=jnp.float32) + b0_ref[...])
    a = _sp(jnp.dot(a, w1_ref[...], preferred_element_type=jnp.float32)
            + b1_ref[...])
    a = jnp.dot(a, w2_ref[...], preferred_element_type=jnp.float32) + b2_ref[...]
    e = jnp.exp(a)
    oh = (bc_ref[...] == lax.broadcasted_iota(
        jnp.int32, (N_NODES, N_GRAPHS), 1)).astype(jnp.float32)
    oht = (br_ref[...] == lax.broadcasted_iota(
        jnp.int32, (N_GRAPHS, N_NODES), 0)).astype(jnp.float32)
    sg = jnp.dot(oht, e, preferred_element_type=jnp.float32)
    sn = jnp.dot(oh, sg, preferred_element_type=jnp.float32)
    attw = e / (sn + 1e-16)
    pooled = jnp.dot(oht, ox * attw, preferred_element_type=jnp.float32)
    hh = _sp(jnp.dot(pooled, pw_ref[...], preferred_element_type=jnp.float32)
             + pb_ref[...])
    out_ref[...] = (jnp.dot(hh, ow_ref[...], preferred_element_type=jnp.float32)
                    + ob_ref[...])


def _tail(ox, gl, batch, w0, b0, w1, b1, w2, b2, pw, pb, ow, ob):
    bc = batch.astype(jnp.int32).reshape(N_NODES, 1)
    br = batch.astype(jnp.int32).reshape(1, N_NODES)
    return pl.pallas_call(
        _tail_body,
        out_shape=jax.ShapeDtypeStruct((N_GRAPHS, 1), jnp.float32),
    )(ox, gl, bc, br, w0[:DIM], w0[DIM:], b0.reshape(1, DIM),
      w1, b1.reshape(1, DIM), w2, b2.reshape(1, 1),
      pw, pb.reshape(1, DIM), ow, ob.reshape(1, 1))


def _sc_gather2(table, idx_i, idx_j):
    E = idx_i.shape[0]
    ch = 80
    nch = EPW // ch

    @functools.partial(
        pl.kernel, mesh=_mesh,
        out_type=[jax.ShapeDtypeStruct((E, HD), jnp.float32),
                  jax.ShapeDtypeStruct((E, HD), jnp.float32)],
        scratch_types=[pltpu.VMEM((ch,), jnp.int32),
                       pltpu.VMEM((ch,), jnp.int32),
                       pltpu.VMEM((ch, HD), jnp.float32),
                       pltpu.VMEM((ch, HD), jnp.float32),
                       pltpu.SemaphoreType.DMA,
                       pltpu.SemaphoreType.DMA])
    def k(tab, ii, ij, gi, gj, iv, jv, ri, rj, s1, s2):
        wid = lax.axis_index("s") * NC + lax.axis_index("c")
        base = wid * EPW

        def body(t, carry):
            o = base + t * ch
            pltpu.sync_copy(ii.at[pl.ds(o, ch)], iv)
            pltpu.sync_copy(ij.at[pl.ds(o, ch)], jv)
            ci = pltpu.async_copy(tab.at[iv], ri, s1)
            cj = pltpu.async_copy(tab.at[jv], rj, s2)
            ci.wait()
            pltpu.sync_copy(ri, gi.at[pl.ds(o, ch)])
            cj.wait()
            pltpu.sync_copy(rj, gj.at[pl.ds(o, ch)])
            return carry

        lax.fori_loop(0, nch, body, 0)

    return k(table, idx_i, idx_j)


def _combine16_body(u_ref, o_ref):
    u = u_ref[...]
    s = u[0] + u[1]
    o_ref[...] = jnp.concatenate(
        [s, jnp.zeros((NPAD, DIM - 16), jnp.float32)], axis=1)


def _combine16(dpart):
    return pl.pallas_call(
        _combine16_body,
        out_shape=jax.ShapeDtypeStruct((NPAD, DIM), jnp.float32),
    )(dpart)


def _sc_gather16(table, idxp):
    E = NW * EPW
    ch = 400
    sub = ch // 80
    nch = EPW // ch

    @functools.partial(
        pl.kernel, mesh=_mesh,
        out_type=jax.ShapeDtypeStruct((E, DIM), jnp.float32),
        compiler_params=pltpu.CompilerParams(use_tc_tiling_on_sc=False),
        scratch_types=[pltpu.VMEM((128, 80), jnp.int32),
                       pltpu.VMEM((ch, DIM), jnp.float32),
                       pltpu.SemaphoreType.DMA])
    def k(t0, ii, dn, iv, rows, s1):
        wid = lax.axis_index("s") * NC + lax.axis_index("c")
        base = wid * EPW
        pltpu.sync_copy(ii.at[pl.ds(wid * 128, 128)], iv)

        for t in range(nch):
            o = base + t * ch
            hs = [pltpu.async_copy(t0.at[iv.at[t * sub + j]],
                                   rows.at[pl.ds(j * 80, 80)], s1)
                  for j in range(sub)]
            for h in hs:
                h.wait()
            pltpu.sync_copy(rows, dn.at[pl.ds(o, ch)])

    return k(table, idxp)


def _sc_scatter(vals, idx_i, d):
    E = idx_i.shape[0]
    ch = 80
    nch = EPW // ch
    rps = NPAD // NS

    @functools.partial(
        pl.kernel, mesh=_mesh,
        out_type=jax.ShapeDtypeStruct((NC, NPAD, d), jnp.float32),
        scratch_types=[pltpu.VMEM((ch,), jnp.int32),
                       pltpu.VMEM((ch, d), jnp.float32),
                       pltpu.VMEM((ch, d), jnp.float32),
                       pltpu.VMEM_SHARED((NPAD, d), jnp.float32)])
    def k(v, ii, out, iv, rows, zb, acc):
        c = lax.axis_index("c")
        s = lax.axis_index("s")
        wid = s * NC + c
        for q in range(ch):
            for j in range(d // 16):
                zb[q, pl.ds(j * 16, 16)] = jnp.zeros((16,), jnp.float32)

        def zbody(t, carry):
            pltpu.sync_copy(zb, acc.at[pl.ds(s * rps + t * ch, ch)])
            return carry

        lax.fori_loop(0, rps // ch, zbody, 0)
        plsc.subcore_barrier()
        base = wid * EPW

        def body(t, carry):
            o = base + t * ch
            pltpu.sync_copy(ii.at[pl.ds(o, ch)], iv)
            pltpu.sync_copy(v.at[pl.ds(o, ch)], rows)
            pltpu.sync_copy(rows, acc.at[iv], add=True)
            return carry

        lax.fori_loop(0, nch, body, 0)
        plsc.subcore_barrier()

        def dbody(t, carry):
            o = s * rps + t * ch
            pltpu.sync_copy(acc.at[pl.ds(o, ch)], zb)
            pltpu.sync_copy(zb, out.at[c, pl.ds(o, ch)])
            return carry

        lax.fori_loop(0, rps // ch, dbody, 0)

    return k(vals, idx_i)


def kernel(x, edge_index, edge_attr, batch, glbl_x, pre_n_w, pre_n_b,
           pre_e_w, pre_e_b, W_stack, att_stack, bias_stack, bn1_g, bn1_b,
           dgn_lin, dgn_g, dgn_b, ga_w0, ga_b0, ga_w1, ga_b1, ga_w2, ga_b2,
           post_w, post_b, out_w, out_b):
    idx_i = edge_index[0].astype(jnp.int32)
    idx_j = edge_index[1].astype(jnp.int32)
    E = idx_i.shape[0]

    def _padidx(ix):
        r = ix.reshape(NW, EPW // 80, 80)
        p = jnp.full((NW, 128 - EPW // 80, 80), NPAD - 1, jnp.int32)
        return jnp.concatenate([r, p], axis=1).reshape(NW * 128, 80)

    idxp_i = _padidx(idx_i)

    gmat = (jnp.arange(128)[:, None] % HEADS
            == jnp.arange(HEADS)[None, :]).astype(jnp.float32)
    gmat_t = gmat.T

    ox, xn = _node_prep(x, pre_n_w, pre_n_b, W_stack[0][:DIM])
    e64 = _edge_prep(edge_attr, pre_e_w, pre_e_b)

    for l in range(GC):
        we = W_stack[l][DIM:]
        attl = att_stack[l][0][:, :DIM].reshape(1, HD)
        attr_ = att_stack[l][0][:, DIM:].reshape(1, HD)
        g128 = jnp.tile(bn1_g[l].reshape(1, HEADS), (1, 128 // HEADS))
        b128 = jnp.tile(bn1_b[l].reshape(1, HEADS), (1, 128 // HEADS))

        gi, gj = _sc_gather2(xn, idx_i, idx_j)
        ap = _passA(gi, gj, e64, we, attl, attr_)
        z128 = _passB(ap.reshape(E * HEADS // 128, 128), g128, b128,
                      gmat, gmat_t)
        z4 = z128.reshape(E, HEADS)
        z16 = jnp.pad(z4, ((0, 0), (0, 16 - HEADS)))
        dpart = _sc_scatter(z16, idx_i, 16)
        dn = _sc_gather16(_combine16(dpart), idxp_i)
        c = _passC(gj, e64, z4, dn, we)
        upart = _sc_scatter(c, idx_i, DIM)
        wxn = W_stack[min(l + 1, GC - 1)][:DIM]
        ox, xn = _passE(upart, bias_stack[l], dgn_lin[l],
                        dgn_g[l].reshape(GROUPS, DIM),
                        dgn_b[l].reshape(GROUPS, DIM), ox, wxn)

    out = _tail(ox, glbl_x, batch, ga_w0, ga_b0, ga_w1, ga_b1, ga_w2, ga_b2,
                post_w, post_b, out_w, out_b)
    return out.reshape(-1)
